# baseline (speedup 1.0000x reference)
import functools

import jax
import jax.numpy as jnp
from jax import lax
from jax.experimental import pallas as pl
from jax.experimental.pallas import tpu as pltpu
from jax.experimental.pallas import tpu_sc as plsc

B, H, HG, E, K, C = 2048, 1024, 512, 8, 2, 3

BMG = 256
LOG2_BMG = 8
NBLK = B * K // BMG + E
NPAD = NBLK * BMG
NW = 32
TPW = B // NW

BM1 = 1024
BM5 = 1024
LANEPAD = 128


def _pack_bf16_pair(xbf):
    n = xbf.shape[1] // 2
    lo = lax.bitcast_convert_type(xbf[:, :n], jnp.uint16).astype(jnp.uint32)
    hi = lax.bitcast_convert_type(xbf[:, n:], jnp.uint16).astype(jnp.uint32)
    return lax.bitcast_convert_type(lo | (hi << 16), jnp.int32)


def _unpack_bf16_pair(xi):
    u = lax.bitcast_convert_type(xi, jnp.uint32)
    lo = lax.bitcast_convert_type((u & 0xFFFF).astype(jnp.uint16), jnp.bfloat16)
    hi = lax.bitcast_convert_type((u >> 16).astype(jnp.uint16), jnp.bfloat16)
    return jnp.concatenate([lo, hi], axis=1)


def _layernorm(h, g, b):
    mu = jnp.mean(h, axis=-1, keepdims=True)
    var = jnp.mean((h - mu) ** 2, axis=-1, keepdims=True)
    return (h - mu) / jnp.sqrt(var + 1e-5) * g + b


def _gating_body(x_ref, gW1_ref, gb1_ref, glng_ref, glnb_ref, gW2_ref, gb2_ref,
                 probs_ref, combine_ref, w1_ref, w2_ref, rt_ref,
                 bg_ref, bvalid_ref, counts_ref, xp_ref, carry_ref):
    i = pl.program_id(0)

    @pl.when(i == 0)
    def _():
        carry_ref[...] = jnp.zeros_like(carry_ref)

    x = x_ref[...]
    xp_ref[...] = _pack_bf16_pair(x.astype(jnp.bfloat16))
    h = jnp.dot(x, gW1_ref[...], preferred_element_type=jnp.float32) + gb1_ref[...]
    h = jax.nn.gelu(_layernorm(h, glng_ref[...], glnb_ref[...]))
    logits = jnp.dot(h, gW2_ref[...], preferred_element_type=jnp.float32) + gb2_ref[...]
    probs = jax.nn.softmax(logits, axis=-1)

    lane = lax.broadcasted_iota(jnp.int32, probs.shape, 1)
    i1 = jnp.argmax(probs, axis=-1)[:, None]
    oh1 = lane == i1
    p2m = jnp.where(oh1, -jnp.inf, probs)
    i2 = jnp.argmax(p2m, axis=-1)[:, None]
    oh2 = lane == i2
    sel = oh1 | oh2
    probs_ref[...] = probs
    combine_ref[...] = jnp.where(sel, probs, 0.0)
    w1_ref[...] = jnp.sum(jnp.where(oh1, probs, 0.0), axis=1, keepdims=True)
    w2_ref[...] = jnp.sum(jnp.where(oh2, probs, 0.0), axis=1, keepdims=True)

    selsum = oh1.astype(jnp.float32) + oh2.astype(jnp.float32)
    r = lax.broadcasted_iota(jnp.int32, (BM1, BM1), 0)
    c = lax.broadcasted_iota(jnp.int32, (BM1, BM1), 1)
    tril = (c < r).astype(jnp.float32)
    excl = jnp.dot(tril, selsum, preferred_element_type=jnp.float32)
    excl = excl + carry_ref[...]
    r1 = jnp.sum(jnp.where(oh1, excl, 0.0), axis=1, keepdims=True).astype(jnp.int32)
    r2 = jnp.sum(jnp.where(oh2, excl, 0.0), axis=1, keepdims=True).astype(jnp.int32)
    rt_ref[...] = jnp.concatenate(
        [jnp.transpose(i1), jnp.transpose(i2),
         jnp.transpose(r1), jnp.transpose(r2)], axis=0)
    new_carry = carry_ref[...] + jnp.sum(selsum, axis=0, keepdims=True)
    carry_ref[...] = new_carry

    counts = new_carry.astype(jnp.int32)
    jj = lax.broadcasted_iota(jnp.int32, (1, LANEPAD), 1)
    bg = jnp.zeros((1, LANEPAD), jnp.int32)
    cpad = jnp.zeros((1, LANEPAD), jnp.int32)
    cum_run = jnp.int32(0)
    for e in range(E):
        cnt_e = jnp.sum(jnp.where(lane[:1] == e, counts, 0))
        cpad = cpad + jnp.where(jj == e, cum_run << LOG2_BMG, 0)
        cum_run = cum_run + ((cnt_e + (BMG - 1)) >> LOG2_BMG)
        bg = bg + (jj >= cum_run).astype(jnp.int32)
    bvalid_ref[...] = (jj < cum_run).astype(jnp.int32)
    bg_ref[...] = jnp.minimum(bg, E - 1)
    counts_ref[...] = cpad


def _dispatch_body(offs_hbm, rt_hbm, x_hbm,
                   xs_hbm, pos1_hbm, pos2_hbm,
                   offs, e1v, e2v, r1v, r2v, p1v, p2v, xbuf, sem, sem2):
    wid = lax.axis_index("s") * 2 + lax.axis_index("c")
    base = wid * TPW
    cp_x = pltpu.async_copy(x_hbm.at[pl.ds(base, TPW)], xbuf, sem)
    pltpu.sync_copy(offs_hbm.at[0, pl.ds(0, 16)], offs)

    pltpu.sync_copy(rt_hbm.at[0, pl.ds(base, TPW)], e1v)
    pltpu.sync_copy(rt_hbm.at[1, pl.ds(base, TPW)], e2v)
    pltpu.sync_copy(rt_hbm.at[2, pl.ds(base, TPW)], r1v)
    pltpu.sync_copy(rt_hbm.at[3, pl.ds(base, TPW)], r2v)
    for c4 in range(TPW // 16):
        sl = pl.ds(c4 * 16, 16)
        e1c, e2c = e1v[sl], e2v[sl]
        ov = offs[...]
        p1 = r1v[sl]
        p2 = r2v[sl]
        for e in range(E):
            off_e = ov[e]
            p1 = p1 + jnp.where(e1c == e, off_e, 0)
            p2 = p2 + jnp.where(e2c == e, off_e, 0)
        p1v[sl] = p1
        p2v[sl] = p2
    pltpu.sync_copy(p1v, pos1_hbm.at[pl.ds(base, TPW)])
    pltpu.sync_copy(p2v, pos2_hbm.at[pl.ds(base, TPW)])

    cp_x.wait()
    c1 = pltpu.async_copy(xbuf, xs_hbm.at[p1v], sem)
    c2 = pltpu.async_copy(xbuf, xs_hbm.at[p2v], sem2)
    c1.wait()
    c2.wait()


def _gmm_body(bg_ref, bvalid_ref, xs_ref, eW_ref, cW1_ref, z_ref, cw1bf_ref):
    j = pl.program_id(0)

    @pl.when(j == 0)
    def _():
        cw1bf_ref[...] = cW1_ref[...].astype(jnp.bfloat16)

    @pl.when(bvalid_ref[0, j] == 1)
    def _():
        xb = _unpack_bf16_pair(xs_ref[...])
        y = jnp.dot(xb, eW_ref[0].astype(jnp.bfloat16),
                    preferred_element_type=jnp.float32)
        z = jnp.dot(y.astype(jnp.bfloat16), cw1bf_ref[...],
                    preferred_element_type=jnp.float32)
        z_ref[...] = _pack_bf16_pair(z.astype(jnp.bfloat16))


def _combine_body(pos1_hbm, pos2_hbm, z_hbm, zg0_hbm, zg1_hbm,
                  idx0, idx1, buf0, buf1, sem, sem2):
    wid = lax.axis_index("s") * 2 + lax.axis_index("c")
    base = wid * TPW
    hw = TPW // 2
    pltpu.sync_copy(pos1_hbm.at[pl.ds(base, TPW)], idx0)
    pltpu.sync_copy(pos2_hbm.at[pl.ds(base, TPW)], idx1)
    cA = pltpu.async_copy(z_hbm.at[idx0.at[pl.ds(0, hw)]], buf0, sem)
    cB = pltpu.async_copy(z_hbm.at[idx0.at[pl.ds(hw, hw)]], buf1, sem2)
    cA.wait()
    pltpu.sync_copy(buf0, zg0_hbm.at[pl.ds(base, hw)])
    cC = pltpu.async_copy(z_hbm.at[idx1.at[pl.ds(0, hw)]], buf0, sem)
    cB.wait()
    pltpu.sync_copy(buf1, zg0_hbm.at[pl.ds(base + hw, hw)])
    cD = pltpu.async_copy(z_hbm.at[idx1.at[pl.ds(hw, hw)]], buf1, sem2)
    cC.wait()
    pltpu.sync_copy(buf0, zg1_hbm.at[pl.ds(base, hw)])
    cD.wait()
    pltpu.sync_copy(buf1, zg1_hbm.at[pl.ds(base + hw, hw)])


def _classifier_body(zg0_ref, zg1_ref, w1_ref, w2_ref, comb_ref, eb_ref,
                     cW1_ref, cb1_ref, clng_ref, clnb_ref, cW2_ref, cb2_ref,
                     out_ref):
    ebW1 = jnp.dot(eb_ref[...], cW1_ref[...], preferred_element_type=jnp.float32)
    zg0 = _unpack_bf16_pair(zg0_ref[...]).astype(jnp.float32)
    zg1 = _unpack_bf16_pair(zg1_ref[...]).astype(jnp.float32)
    u = (zg0 * w1_ref[...] + zg1 * w2_ref[...]
         + jnp.dot(comb_ref[...], ebW1, preferred_element_type=jnp.float32)
         + cb1_ref[...])
    h = jax.nn.relu(_layernorm(u, clng_ref[...], clnb_ref[...]))
    out_ref[...] = jnp.dot(h, cW2_ref[...],
                           preferred_element_type=jnp.float32) + cb2_ref[...]


def _stage2_dispatch(counts, rt, xp):
    mesh = plsc.VectorSubcoreMesh(core_axis_name="c", subcore_axis_name="s")
    dispatch = functools.partial(
        pl.kernel, mesh=mesh,
        out_type=[
            jax.ShapeDtypeStruct((NPAD, H // 2), jnp.int32),
            jax.ShapeDtypeStruct((B,), jnp.int32),
            jax.ShapeDtypeStruct((B,), jnp.int32),
        ],
        scratch_types=[
            pltpu.VMEM((16,), jnp.int32),
            pltpu.VMEM((TPW,), jnp.int32),
            pltpu.VMEM((TPW,), jnp.int32),
            pltpu.VMEM((TPW,), jnp.int32),
            pltpu.VMEM((TPW,), jnp.int32),
            pltpu.VMEM((TPW,), jnp.int32),
            pltpu.VMEM((TPW,), jnp.int32),
            pltpu.VMEM((TPW, H // 2), jnp.int32),
            pltpu.SemaphoreType.DMA,
            pltpu.SemaphoreType.DMA,
        ],
    )(_dispatch_body)
    return dispatch(counts, rt, xp)


def _stage4_combine(pos1, pos2, z):
    mesh = plsc.VectorSubcoreMesh(core_axis_name="c", subcore_axis_name="s")
    combgather = functools.partial(
        pl.kernel, mesh=mesh,
        out_type=[
            jax.ShapeDtypeStruct((B, HG // 2), jnp.int32),
            jax.ShapeDtypeStruct((B, HG // 2), jnp.int32),
        ],
        scratch_types=[
            pltpu.VMEM((TPW,), jnp.int32),
            pltpu.VMEM((TPW,), jnp.int32),
            pltpu.VMEM((TPW // 2, HG // 2), jnp.int32),
            pltpu.VMEM((TPW // 2, HG // 2), jnp.int32),
            pltpu.SemaphoreType.DMA,
            pltpu.SemaphoreType.DMA,
        ],
    )(_combine_body)
    return combgather(pos1, pos2, z)


def kernel(x, gW1, gb1, gln_g, gln_b, gW2, gb2, eW, eb, cW1, cb1, cln_g, cln_b, cW2, cb2):
    nb1 = B // BM1
    full = lambda shape: pl.BlockSpec(shape, lambda i: (0,) * len(shape))

    (probs, combine, w1, w2, rt, bg, bvalid, counts, xp) = pl.pallas_call(
        _gating_body,
        grid=(nb1,),
        in_specs=[
            pl.BlockSpec((BM1, H), lambda i: (i, 0)),
            full((H, HG)), full((1, HG)), full((1, HG)), full((1, HG)),
            full((HG, E)), full((1, E)),
        ],
        out_specs=[
            pl.BlockSpec((BM1, E), lambda i: (i, 0)),
            pl.BlockSpec((BM1, E), lambda i: (i, 0)),
            pl.BlockSpec((BM1, 1), lambda i: (i, 0)),
            pl.BlockSpec((BM1, 1), lambda i: (i, 0)),
            pl.BlockSpec((4, BM1), lambda i: (0, i)),
            full((1, LANEPAD)), full((1, LANEPAD)), full((1, LANEPAD)),
            pl.BlockSpec((BM1, H // 2), lambda i: (i, 0)),
        ],
        out_shape=[
            jax.ShapeDtypeStruct((B, E), jnp.float32),
            jax.ShapeDtypeStruct((B, E), jnp.float32),
            jax.ShapeDtypeStruct((B, 1), jnp.float32),
            jax.ShapeDtypeStruct((B, 1), jnp.float32),
            jax.ShapeDtypeStruct((4, B), jnp.int32),
            jax.ShapeDtypeStruct((1, LANEPAD), jnp.int32),
            jax.ShapeDtypeStruct((1, LANEPAD), jnp.int32),
            jax.ShapeDtypeStruct((1, LANEPAD), jnp.int32),
            jax.ShapeDtypeStruct((B, H // 2), jnp.int32),
        ],
        scratch_shapes=[pltpu.VMEM((1, E), jnp.float32)],
        compiler_params=pltpu.CompilerParams(
            dimension_semantics=("arbitrary",)),
    )(x, gW1, gb1.reshape(1, HG), gln_g.reshape(1, HG), gln_b.reshape(1, HG),
      gW2, gb2.reshape(1, E))

    xs, pos1, pos2 = _stage2_dispatch(counts, rt, xp)

    z = pl.pallas_call(
        _gmm_body,
        grid_spec=pltpu.PrefetchScalarGridSpec(
            num_scalar_prefetch=2,
            grid=(NBLK,),
            in_specs=[
                pl.BlockSpec((BMG, H // 2), lambda j, bg, bv: (bv[0, j] * j, 0)),
                pl.BlockSpec((1, H, H), lambda j, bg, bv: (bg[0, j], 0, 0)),
                pl.BlockSpec((H, HG), lambda j, bg, bv: (0, 0)),
            ],
            out_specs=pl.BlockSpec(
                (BMG, HG // 2), lambda j, bg, bv: (jnp.where(bv[0, j] == 1, j, NBLK), 0)),
            scratch_shapes=[pltpu.VMEM((H, HG), jnp.bfloat16)],
        ),
        out_shape=jax.ShapeDtypeStruct(((NBLK + 1) * BMG, HG // 2), jnp.int32),
        compiler_params=pltpu.CompilerParams(
            dimension_semantics=("arbitrary",)),
    )(bg, bvalid, xs, eW, cW1)

    zg0, zg1 = _stage4_combine(pos1, pos2, z)

    nb5 = B // BM5
    logits = pl.pallas_call(
        _classifier_body,
        grid=(nb5,),
        in_specs=[
            pl.BlockSpec((BM5, HG // 2), lambda i: (i, 0)),
            pl.BlockSpec((BM5, HG // 2), lambda i: (i, 0)),
            pl.BlockSpec((BM5, 1), lambda i: (i, 0)),
            pl.BlockSpec((BM5, 1), lambda i: (i, 0)),
            pl.BlockSpec((BM5, E), lambda i: (i, 0)),
            full((E, H)),
            full((H, HG)), full((1, HG)), full((1, HG)), full((1, HG)),
            full((HG, C)), full((1, C)),
        ],
        out_specs=pl.BlockSpec((BM5, C), lambda i: (i, 0)),
        out_shape=jax.ShapeDtypeStruct((B, C), jnp.float32),
        compiler_params=pltpu.CompilerParams(
            dimension_semantics=("parallel",)),
    )(zg0, zg1, w1, w2, combine, eb,
      cW1, cb1.reshape(1, HG), cln_g.reshape(1, HG), cln_b.reshape(1, HG),
      cW2, cb2.reshape(1, C))

    return logits, probs


# speedup vs baseline: 1.0293x; 1.0293x over previous
import functools

import jax
import jax.numpy as jnp
from jax import lax
from jax.experimental import pallas as pl
from jax.experimental.pallas import tpu as pltpu
from jax.experimental.pallas import tpu_sc as plsc

B, H, HG, E, K, C = 2048, 1024, 512, 8, 2, 3

BMG = 512
LOG2_BMG = 9
NBLK = B * K // BMG + E
NPAD = NBLK * BMG
NW = 32
TPW = B // NW

BM1 = 1024
BM5 = 1024
LANEPAD = 128


def _pack_bf16_pair(xbf):
    n = xbf.shape[1] // 2
    lo = lax.bitcast_convert_type(xbf[:, :n], jnp.uint16).astype(jnp.uint32)
    hi = lax.bitcast_convert_type(xbf[:, n:], jnp.uint16).astype(jnp.uint32)
    return lax.bitcast_convert_type(lo | (hi << 16), jnp.int32)


def _unpack_bf16_pair(xi):
    u = lax.bitcast_convert_type(xi, jnp.uint32)
    lo = lax.bitcast_convert_type((u & 0xFFFF).astype(jnp.uint16), jnp.bfloat16)
    hi = lax.bitcast_convert_type((u >> 16).astype(jnp.uint16), jnp.bfloat16)
    return jnp.concatenate([lo, hi], axis=1)


def _layernorm(h, g, b):
    mu = jnp.mean(h, axis=-1, keepdims=True)
    var = jnp.mean((h - mu) ** 2, axis=-1, keepdims=True)
    return (h - mu) / jnp.sqrt(var + 1e-5) * g + b


def _gating_body(x_ref, gW1_ref, gb1_ref, glng_ref, glnb_ref, gW2_ref, gb2_ref,
                 probs_ref, combine_ref, w1_ref, w2_ref, rt_ref,
                 bg_ref, bvalid_ref, counts_ref, xp_ref, carry_ref):
    i = pl.program_id(0)

    @pl.when(i == 0)
    def _():
        carry_ref[...] = jnp.zeros_like(carry_ref)

    x = x_ref[...]
    xp_ref[...] = _pack_bf16_pair(x.astype(jnp.bfloat16))
    h = jnp.dot(x, gW1_ref[...], preferred_element_type=jnp.float32) + gb1_ref[...]
    h = jax.nn.gelu(_layernorm(h, glng_ref[...], glnb_ref[...]))
    logits = jnp.dot(h, gW2_ref[...], preferred_element_type=jnp.float32) + gb2_ref[...]
    probs = jax.nn.softmax(logits, axis=-1)

    lane = lax.broadcasted_iota(jnp.int32, probs.shape, 1)
    i1 = jnp.argmax(probs, axis=-1)[:, None]
    oh1 = lane == i1
    p2m = jnp.where(oh1, -jnp.inf, probs)
    i2 = jnp.argmax(p2m, axis=-1)[:, None]
    oh2 = lane == i2
    sel = oh1 | oh2
    probs_ref[...] = probs
    combine_ref[...] = jnp.where(sel, probs, 0.0)
    w1_ref[...] = jnp.sum(jnp.where(oh1, probs, 0.0), axis=1, keepdims=True)
    w2_ref[...] = jnp.sum(jnp.where(oh2, probs, 0.0), axis=1, keepdims=True)

    selsum = oh1.astype(jnp.float32) + oh2.astype(jnp.float32)
    r = lax.broadcasted_iota(jnp.int32, (BM1, BM1), 0)
    c = lax.broadcasted_iota(jnp.int32, (BM1, BM1), 1)
    tril = (c < r).astype(jnp.float32)
    excl = jnp.dot(tril, selsum, preferred_element_type=jnp.float32)
    excl = excl + carry_ref[...]
    r1 = jnp.sum(jnp.where(oh1, excl, 0.0), axis=1, keepdims=True).astype(jnp.int32)
    r2 = jnp.sum(jnp.where(oh2, excl, 0.0), axis=1, keepdims=True).astype(jnp.int32)
    rt_ref[...] = jnp.concatenate(
        [jnp.transpose(i1), jnp.transpose(i2),
         jnp.transpose(r1), jnp.transpose(r2)], axis=0)
    new_carry = carry_ref[...] + jnp.sum(selsum, axis=0, keepdims=True)
    carry_ref[...] = new_carry

    counts = new_carry.astype(jnp.int32)
    jj = lax.broadcasted_iota(jnp.int32, (1, LANEPAD), 1)
    bg = jnp.zeros((1, LANEPAD), jnp.int32)
    cpad = jnp.zeros((1, LANEPAD), jnp.int32)
    cum_run = jnp.int32(0)
    for e in range(E):
        cnt_e = jnp.sum(jnp.where(lane[:1] == e, counts, 0))
        cpad = cpad + jnp.where(jj == e, cum_run << LOG2_BMG, 0)
        cum_run = cum_run + ((cnt_e + (BMG - 1)) >> LOG2_BMG)
        bg = bg + (jj >= cum_run).astype(jnp.int32)
    bvalid_ref[...] = (jj < cum_run).astype(jnp.int32)
    bg_ref[...] = jnp.minimum(bg, E - 1)
    counts_ref[...] = cpad


def _dispatch_body(offs_hbm, rt_hbm, x_hbm,
                   xs_hbm, pos1_hbm, pos2_hbm,
                   offs, e1v, e2v, r1v, r2v, p1v, p2v, xbuf, sem, sem2):
    wid = lax.axis_index("s") * 2 + lax.axis_index("c")
    base = wid * TPW
    cp_x = pltpu.async_copy(x_hbm.at[pl.ds(base, TPW)], xbuf, sem)
    pltpu.sync_copy(offs_hbm.at[0, pl.ds(0, 16)], offs)

    pltpu.sync_copy(rt_hbm.at[0, pl.ds(base, TPW)], e1v)
    pltpu.sync_copy(rt_hbm.at[1, pl.ds(base, TPW)], e2v)
    pltpu.sync_copy(rt_hbm.at[2, pl.ds(base, TPW)], r1v)
    pltpu.sync_copy(rt_hbm.at[3, pl.ds(base, TPW)], r2v)
    for c4 in range(TPW // 16):
        sl = pl.ds(c4 * 16, 16)
        e1c, e2c = e1v[sl], e2v[sl]
        ov = offs[...]
        p1 = r1v[sl]
        p2 = r2v[sl]
        for e in range(E):
            off_e = ov[e]
            p1 = p1 + jnp.where(e1c == e, off_e, 0)
            p2 = p2 + jnp.where(e2c == e, off_e, 0)
        p1v[sl] = p1
        p2v[sl] = p2
    pltpu.sync_copy(p1v, pos1_hbm.at[pl.ds(base, TPW)])
    pltpu.sync_copy(p2v, pos2_hbm.at[pl.ds(base, TPW)])

    cp_x.wait()
    c1 = pltpu.async_copy(xbuf, xs_hbm.at[p1v], sem)
    c2 = pltpu.async_copy(xbuf, xs_hbm.at[p2v], sem2)
    c1.wait()
    c2.wait()


def _gmm_body(bg_ref, bvalid_ref, xs_ref, eW_ref, cW1_ref, z_ref, cw1bf_ref):
    j = pl.program_id(0)

    @pl.when(j == 0)
    def _():
        cw1bf_ref[...] = cW1_ref[...].astype(jnp.bfloat16)

    @pl.when(bvalid_ref[0, j] == 1)
    def _():
        xb = _unpack_bf16_pair(xs_ref[...])
        y = jnp.dot(xb, eW_ref[0].astype(jnp.bfloat16),
                    preferred_element_type=jnp.float32)
        z = jnp.dot(y.astype(jnp.bfloat16), cw1bf_ref[...],
                    preferred_element_type=jnp.float32)
        z_ref[...] = _pack_bf16_pair(z.astype(jnp.bfloat16))


def _combine_body(pos1_hbm, pos2_hbm, z_hbm, zg0_hbm, zg1_hbm,
                  idx0, idx1, buf0, buf1, sem, sem2):
    wid = lax.axis_index("s") * 2 + lax.axis_index("c")
    base = wid * TPW
    hw = TPW // 2
    pltpu.sync_copy(pos1_hbm.at[pl.ds(base, TPW)], idx0)
    pltpu.sync_copy(pos2_hbm.at[pl.ds(base, TPW)], idx1)
    cA = pltpu.async_copy(z_hbm.at[idx0.at[pl.ds(0, hw)]], buf0, sem)
    cB = pltpu.async_copy(z_hbm.at[idx0.at[pl.ds(hw, hw)]], buf1, sem2)
    cA.wait()
    pltpu.sync_copy(buf0, zg0_hbm.at[pl.ds(base, hw)])
    cC = pltpu.async_copy(z_hbm.at[idx1.at[pl.ds(0, hw)]], buf0, sem)
    cB.wait()
    pltpu.sync_copy(buf1, zg0_hbm.at[pl.ds(base + hw, hw)])
    cD = pltpu.async_copy(z_hbm.at[idx1.at[pl.ds(hw, hw)]], buf1, sem2)
    cC.wait()
    pltpu.sync_copy(buf0, zg1_hbm.at[pl.ds(base, hw)])
    cD.wait()
    pltpu.sync_copy(buf1, zg1_hbm.at[pl.ds(base + hw, hw)])


def _classifier_body(zg0_ref, zg1_ref, w1_ref, w2_ref, comb_ref, eb_ref,
                     cW1_ref, cb1_ref, clng_ref, clnb_ref, cW2_ref, cb2_ref,
                     out_ref):
    ebW1 = jnp.dot(eb_ref[...], cW1_ref[...], preferred_element_type=jnp.float32)
    zg0 = _unpack_bf16_pair(zg0_ref[...]).astype(jnp.float32)
    zg1 = _unpack_bf16_pair(zg1_ref[...]).astype(jnp.float32)
    u = (zg0 * w1_ref[...] + zg1 * w2_ref[...]
         + jnp.dot(comb_ref[...], ebW1, preferred_element_type=jnp.float32)
         + cb1_ref[...])
    h = jax.nn.relu(_layernorm(u, clng_ref[...], clnb_ref[...]))
    out_ref[...] = jnp.dot(h, cW2_ref[...],
                           preferred_element_type=jnp.float32) + cb2_ref[...]


def _stage2_dispatch(counts, rt, xp):
    mesh = plsc.VectorSubcoreMesh(core_axis_name="c", subcore_axis_name="s")
    dispatch = functools.partial(
        pl.kernel, mesh=mesh,
        out_type=[
            jax.ShapeDtypeStruct((NPAD, H // 2), jnp.int32),
            jax.ShapeDtypeStruct((B,), jnp.int32),
            jax.ShapeDtypeStruct((B,), jnp.int32),
        ],
        scratch_types=[
            pltpu.VMEM((16,), jnp.int32),
            pltpu.VMEM((TPW,), jnp.int32),
            pltpu.VMEM((TPW,), jnp.int32),
            pltpu.VMEM((TPW,), jnp.int32),
            pltpu.VMEM((TPW,), jnp.int32),
            pltpu.VMEM((TPW,), jnp.int32),
            pltpu.VMEM((TPW,), jnp.int32),
            pltpu.VMEM((TPW, H // 2), jnp.int32),
            pltpu.SemaphoreType.DMA,
            pltpu.SemaphoreType.DMA,
        ],
    )(_dispatch_body)
    return dispatch(counts, rt, xp)


def _stage4_combine(pos1, pos2, z):
    mesh = plsc.VectorSubcoreMesh(core_axis_name="c", subcore_axis_name="s")
    combgather = functools.partial(
        pl.kernel, mesh=mesh,
        out_type=[
            jax.ShapeDtypeStruct((B, HG // 2), jnp.int32),
            jax.ShapeDtypeStruct((B, HG // 2), jnp.int32),
        ],
        scratch_types=[
            pltpu.VMEM((TPW,), jnp.int32),
            pltpu.VMEM((TPW,), jnp.int32),
            pltpu.VMEM((TPW // 2, HG // 2), jnp.int32),
            pltpu.VMEM((TPW // 2, HG // 2), jnp.int32),
            pltpu.SemaphoreType.DMA,
            pltpu.SemaphoreType.DMA,
        ],
    )(_combine_body)
    return combgather(pos1, pos2, z)


def kernel(x, gW1, gb1, gln_g, gln_b, gW2, gb2, eW, eb, cW1, cb1, cln_g, cln_b, cW2, cb2):
    nb1 = B // BM1
    full = lambda shape: pl.BlockSpec(shape, lambda i: (0,) * len(shape))

    (probs, combine, w1, w2, rt, bg, bvalid, counts, xp) = pl.pallas_call(
        _gating_body,
        grid=(nb1,),
        in_specs=[
            pl.BlockSpec((BM1, H), lambda i: (i, 0)),
            full((H, HG)), full((1, HG)), full((1, HG)), full((1, HG)),
            full((HG, E)), full((1, E)),
        ],
        out_specs=[
            pl.BlockSpec((BM1, E), lambda i: (i, 0)),
            pl.BlockSpec((BM1, E), lambda i: (i, 0)),
            pl.BlockSpec((BM1, 1), lambda i: (i, 0)),
            pl.BlockSpec((BM1, 1), lambda i: (i, 0)),
            pl.BlockSpec((4, BM1), lambda i: (0, i)),
            full((1, LANEPAD)), full((1, LANEPAD)), full((1, LANEPAD)),
            pl.BlockSpec((BM1, H // 2), lambda i: (i, 0)),
        ],
        out_shape=[
            jax.ShapeDtypeStruct((B, E), jnp.float32),
            jax.ShapeDtypeStruct((B, E), jnp.float32),
            jax.ShapeDtypeStruct((B, 1), jnp.float32),
            jax.ShapeDtypeStruct((B, 1), jnp.float32),
            jax.ShapeDtypeStruct((4, B), jnp.int32),
            jax.ShapeDtypeStruct((1, LANEPAD), jnp.int32),
            jax.ShapeDtypeStruct((1, LANEPAD), jnp.int32),
            jax.ShapeDtypeStruct((1, LANEPAD), jnp.int32),
            jax.ShapeDtypeStruct((B, H // 2), jnp.int32),
        ],
        scratch_shapes=[pltpu.VMEM((1, E), jnp.float32)],
        compiler_params=pltpu.CompilerParams(
            dimension_semantics=("arbitrary",)),
    )(x, gW1, gb1.reshape(1, HG), gln_g.reshape(1, HG), gln_b.reshape(1, HG),
      gW2, gb2.reshape(1, E))

    xs, pos1, pos2 = _stage2_dispatch(counts, rt, xp)

    z = pl.pallas_call(
        _gmm_body,
        grid_spec=pltpu.PrefetchScalarGridSpec(
            num_scalar_prefetch=2,
            grid=(NBLK,),
            in_specs=[
                pl.BlockSpec((BMG, H // 2), lambda j, bg, bv: (bv[0, j] * j, 0)),
                pl.BlockSpec((1, H, H), lambda j, bg, bv: (bg[0, j], 0, 0)),
                pl.BlockSpec((H, HG), lambda j, bg, bv: (0, 0)),
            ],
            out_specs=pl.BlockSpec(
                (BMG, HG // 2), lambda j, bg, bv: (jnp.where(bv[0, j] == 1, j, NBLK), 0)),
            scratch_shapes=[pltpu.VMEM((H, HG), jnp.bfloat16)],
        ),
        out_shape=jax.ShapeDtypeStruct(((NBLK + 1) * BMG, HG // 2), jnp.int32),
        compiler_params=pltpu.CompilerParams(
            dimension_semantics=("arbitrary",)),
    )(bg, bvalid, xs, eW, cW1)

    zg0, zg1 = _stage4_combine(pos1, pos2, z)

    nb5 = B // BM5
    logits = pl.pallas_call(
        _classifier_body,
        grid=(nb5,),
        in_specs=[
            pl.BlockSpec((BM5, HG // 2), lambda i: (i, 0)),
            pl.BlockSpec((BM5, HG // 2), lambda i: (i, 0)),
            pl.BlockSpec((BM5, 1), lambda i: (i, 0)),
            pl.BlockSpec((BM5, 1), lambda i: (i, 0)),
            pl.BlockSpec((BM5, E), lambda i: (i, 0)),
            full((E, H)),
            full((H, HG)), full((1, HG)), full((1, HG)), full((1, HG)),
            full((HG, C)), full((1, C)),
        ],
        out_specs=pl.BlockSpec((BM5, C), lambda i: (i, 0)),
        out_shape=jax.ShapeDtypeStruct((B, C), jnp.float32),
        compiler_params=pltpu.CompilerParams(
            dimension_semantics=("parallel",)),
    )(zg0, zg1, w1, w2, combine, eb,
      cW1, cb1.reshape(1, HG), cln_g.reshape(1, HG), cln_b.reshape(1, HG),
      cW2, cb2.reshape(1, C))

    return logits, probs


# speedup vs baseline: 1.0374x; 1.0078x over previous
import functools

import jax
import jax.numpy as jnp
from jax import lax
from jax.experimental import pallas as pl
from jax.experimental.pallas import tpu as pltpu
from jax.experimental.pallas import tpu_sc as plsc

B, H, HG, E, K, C = 2048, 1024, 512, 8, 2, 3

BMG = 512
LOG2_BMG = 9
NBLK = B * K // BMG + E
NPAD = NBLK * BMG
NW = 32
TPW = B // NW

BM1 = 512
BM5 = 1024
LANEPAD = 128


def _pack_bf16_pair(xbf):
    n = xbf.shape[1] // 2
    lo = lax.bitcast_convert_type(xbf[:, :n], jnp.uint16).astype(jnp.uint32)
    hi = lax.bitcast_convert_type(xbf[:, n:], jnp.uint16).astype(jnp.uint32)
    return lax.bitcast_convert_type(lo | (hi << 16), jnp.int32)


def _unpack_bf16_pair(xi):
    u = lax.bitcast_convert_type(xi, jnp.uint32)
    lo = lax.bitcast_convert_type((u & 0xFFFF).astype(jnp.uint16), jnp.bfloat16)
    hi = lax.bitcast_convert_type((u >> 16).astype(jnp.uint16), jnp.bfloat16)
    return jnp.concatenate([lo, hi], axis=1)


def _layernorm(h, g, b):
    mu = jnp.mean(h, axis=-1, keepdims=True)
    var = jnp.mean((h - mu) ** 2, axis=-1, keepdims=True)
    return (h - mu) / jnp.sqrt(var + 1e-5) * g + b


def _gating_body(x_ref, gW1_ref, gb1_ref, glng_ref, glnb_ref, gW2_ref, gb2_ref,
                 probs_ref, combine_ref, w1_ref, w2_ref, rt_ref,
                 bg_ref, bvalid_ref, counts_ref, xp_ref, carry_ref):
    i = pl.program_id(0)

    @pl.when(i == 0)
    def _():
        carry_ref[...] = jnp.zeros_like(carry_ref)

    x = x_ref[...]
    xp_ref[...] = _pack_bf16_pair(x.astype(jnp.bfloat16))
    h = jnp.dot(x, gW1_ref[...], preferred_element_type=jnp.float32) + gb1_ref[...]
    h = jax.nn.gelu(_layernorm(h, glng_ref[...], glnb_ref[...]))
    logits = jnp.dot(h, gW2_ref[...], preferred_element_type=jnp.float32) + gb2_ref[...]
    probs = jax.nn.softmax(logits, axis=-1)

    lane = lax.broadcasted_iota(jnp.int32, probs.shape, 1)
    i1 = jnp.argmax(probs, axis=-1)[:, None]
    oh1 = lane == i1
    p2m = jnp.where(oh1, -jnp.inf, probs)
    i2 = jnp.argmax(p2m, axis=-1)[:, None]
    oh2 = lane == i2
    sel = oh1 | oh2
    probs_ref[...] = probs
    combine_ref[...] = jnp.where(sel, probs, 0.0)
    w1_ref[...] = jnp.sum(jnp.where(oh1, probs, 0.0), axis=1, keepdims=True)
    w2_ref[...] = jnp.sum(jnp.where(oh2, probs, 0.0), axis=1, keepdims=True)

    selsum = oh1.astype(jnp.float32) + oh2.astype(jnp.float32)
    r = lax.broadcasted_iota(jnp.int32, (BM1, BM1), 0)
    c = lax.broadcasted_iota(jnp.int32, (BM1, BM1), 1)
    tril = (c < r).astype(jnp.float32)
    excl = jnp.dot(tril, selsum, preferred_element_type=jnp.float32)
    excl = excl + carry_ref[...]
    r1 = jnp.sum(jnp.where(oh1, excl, 0.0), axis=1, keepdims=True).astype(jnp.int32)
    r2 = jnp.sum(jnp.where(oh2, excl, 0.0), axis=1, keepdims=True).astype(jnp.int32)
    rt_ref[...] = jnp.concatenate(
        [jnp.transpose(i1), jnp.transpose(i2),
         jnp.transpose(r1), jnp.transpose(r2)], axis=0)
    new_carry = carry_ref[...] + jnp.sum(selsum, axis=0, keepdims=True)
    carry_ref[...] = new_carry

    counts = new_carry.astype(jnp.int32)
    jj = lax.broadcasted_iota(jnp.int32, (1, LANEPAD), 1)
    bg = jnp.zeros((1, LANEPAD), jnp.int32)
    cpad = jnp.zeros((1, LANEPAD), jnp.int32)
    cum_run = jnp.int32(0)
    for e in range(E):
        cnt_e = jnp.sum(jnp.where(lane[:1] == e, counts, 0))
        cpad = cpad + jnp.where(jj == e, cum_run << LOG2_BMG, 0)
        cum_run = cum_run + ((cnt_e + (BMG - 1)) >> LOG2_BMG)
        bg = bg + (jj >= cum_run).astype(jnp.int32)
    bvalid_ref[...] = (jj < cum_run).astype(jnp.int32)
    bg_ref[...] = jnp.minimum(bg, E - 1)
    counts_ref[...] = cpad


def _dispatch_body(offs_hbm, rt_hbm, x_hbm,
                   xs_hbm, pos1_hbm, pos2_hbm,
                   offs, e1v, e2v, r1v, r2v, p1v, p2v, xbuf, sem, sem2):
    wid = lax.axis_index("s") * 2 + lax.axis_index("c")
    base = wid * TPW
    cp_x = pltpu.async_copy(x_hbm.at[pl.ds(base, TPW)], xbuf, sem)
    pltpu.sync_copy(offs_hbm.at[0, pl.ds(0, 16)], offs)

    pltpu.sync_copy(rt_hbm.at[0, pl.ds(base, TPW)], e1v)
    pltpu.sync_copy(rt_hbm.at[1, pl.ds(base, TPW)], e2v)
    pltpu.sync_copy(rt_hbm.at[2, pl.ds(base, TPW)], r1v)
    pltpu.sync_copy(rt_hbm.at[3, pl.ds(base, TPW)], r2v)
    for c4 in range(TPW // 16):
        sl = pl.ds(c4 * 16, 16)
        e1c, e2c = e1v[sl], e2v[sl]
        ov = offs[...]
        p1 = r1v[sl]
        p2 = r2v[sl]
        for e in range(E):
            off_e = ov[e]
            p1 = p1 + jnp.where(e1c == e, off_e, 0)
            p2 = p2 + jnp.where(e2c == e, off_e, 0)
        p1v[sl] = p1
        p2v[sl] = p2
    pltpu.sync_copy(p1v, pos1_hbm.at[pl.ds(base, TPW)])
    pltpu.sync_copy(p2v, pos2_hbm.at[pl.ds(base, TPW)])

    cp_x.wait()
    c1 = pltpu.async_copy(xbuf, xs_hbm.at[p1v], sem)
    c2 = pltpu.async_copy(xbuf, xs_hbm.at[p2v], sem2)
    c1.wait()
    c2.wait()


def _gmm_body(bg_ref, bvalid_ref, xs_ref, eW_ref, cW1_ref, z_ref, cw1bf_ref):
    j = pl.program_id(0)

    @pl.when(j == 0)
    def _():
        cw1bf_ref[...] = cW1_ref[...].astype(jnp.bfloat16)

    @pl.when(bvalid_ref[0, j] == 1)
    def _():
        xb = _unpack_bf16_pair(xs_ref[...])
        y = jnp.dot(xb, eW_ref[0].astype(jnp.bfloat16),
                    preferred_element_type=jnp.float32)
        z = jnp.dot(y.astype(jnp.bfloat16), cw1bf_ref[...],
                    preferred_element_type=jnp.float32)
        z_ref[...] = _pack_bf16_pair(z.astype(jnp.bfloat16))


def _combine_body(pos1_hbm, pos2_hbm, z_hbm, zg0_hbm, zg1_hbm,
                  idx0, idx1, buf0, buf1, sem, sem2):
    wid = lax.axis_index("s") * 2 + lax.axis_index("c")
    base = wid * TPW
    hw = TPW // 2
    pltpu.sync_copy(pos1_hbm.at[pl.ds(base, TPW)], idx0)
    pltpu.sync_copy(pos2_hbm.at[pl.ds(base, TPW)], idx1)
    cA = pltpu.async_copy(z_hbm.at[idx0.at[pl.ds(0, hw)]], buf0, sem)
    cB = pltpu.async_copy(z_hbm.at[idx0.at[pl.ds(hw, hw)]], buf1, sem2)
    cA.wait()
    pltpu.sync_copy(buf0, zg0_hbm.at[pl.ds(base, hw)])
    cC = pltpu.async_copy(z_hbm.at[idx1.at[pl.ds(0, hw)]], buf0, sem)
    cB.wait()
    pltpu.sync_copy(buf1, zg0_hbm.at[pl.ds(base + hw, hw)])
    cD = pltpu.async_copy(z_hbm.at[idx1.at[pl.ds(hw, hw)]], buf1, sem2)
    cC.wait()
    pltpu.sync_copy(buf0, zg1_hbm.at[pl.ds(base, hw)])
    cD.wait()
    pltpu.sync_copy(buf1, zg1_hbm.at[pl.ds(base + hw, hw)])


def _classifier_body(zg0_ref, zg1_ref, w1_ref, w2_ref, comb_ref, eb_ref,
                     cW1_ref, cb1_ref, clng_ref, clnb_ref, cW2_ref, cb2_ref,
                     out_ref):
    ebW1 = jnp.dot(eb_ref[...], cW1_ref[...], preferred_element_type=jnp.float32)
    zg0 = _unpack_bf16_pair(zg0_ref[...]).astype(jnp.float32)
    zg1 = _unpack_bf16_pair(zg1_ref[...]).astype(jnp.float32)
    u = (zg0 * w1_ref[...] + zg1 * w2_ref[...]
         + jnp.dot(comb_ref[...], ebW1, preferred_element_type=jnp.float32)
         + cb1_ref[...])
    h = jax.nn.relu(_layernorm(u, clng_ref[...], clnb_ref[...]))
    out_ref[...] = jnp.dot(h, cW2_ref[...],
                           preferred_element_type=jnp.float32) + cb2_ref[...]


def _stage2_dispatch(counts, rt, xp):
    mesh = plsc.VectorSubcoreMesh(core_axis_name="c", subcore_axis_name="s")
    dispatch = functools.partial(
        pl.kernel, mesh=mesh,
        out_type=[
            jax.ShapeDtypeStruct((NPAD, H // 2), jnp.int32),
            jax.ShapeDtypeStruct((B,), jnp.int32),
            jax.ShapeDtypeStruct((B,), jnp.int32),
        ],
        scratch_types=[
            pltpu.VMEM((16,), jnp.int32),
            pltpu.VMEM((TPW,), jnp.int32),
            pltpu.VMEM((TPW,), jnp.int32),
            pltpu.VMEM((TPW,), jnp.int32),
            pltpu.VMEM((TPW,), jnp.int32),
            pltpu.VMEM((TPW,), jnp.int32),
            pltpu.VMEM((TPW,), jnp.int32),
            pltpu.VMEM((TPW, H // 2), jnp.int32),
            pltpu.SemaphoreType.DMA,
            pltpu.SemaphoreType.DMA,
        ],
    )(_dispatch_body)
    return dispatch(counts, rt, xp)


def _stage4_combine(pos1, pos2, z):
    mesh = plsc.VectorSubcoreMesh(core_axis_name="c", subcore_axis_name="s")
    combgather = functools.partial(
        pl.kernel, mesh=mesh,
        out_type=[
            jax.ShapeDtypeStruct((B, HG // 2), jnp.int32),
            jax.ShapeDtypeStruct((B, HG // 2), jnp.int32),
        ],
        scratch_types=[
            pltpu.VMEM((TPW,), jnp.int32),
            pltpu.VMEM((TPW,), jnp.int32),
            pltpu.VMEM((TPW // 2, HG // 2), jnp.int32),
            pltpu.VMEM((TPW // 2, HG // 2), jnp.int32),
            pltpu.SemaphoreType.DMA,
            pltpu.SemaphoreType.DMA,
        ],
    )(_combine_body)
    return combgather(pos1, pos2, z)


def kernel(x, gW1, gb1, gln_g, gln_b, gW2, gb2, eW, eb, cW1, cb1, cln_g, cln_b, cW2, cb2):
    nb1 = B // BM1
    full = lambda shape: pl.BlockSpec(shape, lambda i: (0,) * len(shape))

    (probs, combine, w1, w2, rt, bg, bvalid, counts, xp) = pl.pallas_call(
        _gating_body,
        grid=(nb1,),
        in_specs=[
            pl.BlockSpec((BM1, H), lambda i: (i, 0)),
            full((H, HG)), full((1, HG)), full((1, HG)), full((1, HG)),
            full((HG, E)), full((1, E)),
        ],
        out_specs=[
            pl.BlockSpec((BM1, E), lambda i: (i, 0)),
            pl.BlockSpec((BM1, E), lambda i: (i, 0)),
            pl.BlockSpec((BM1, 1), lambda i: (i, 0)),
            pl.BlockSpec((BM1, 1), lambda i: (i, 0)),
            pl.BlockSpec((4, BM1), lambda i: (0, i)),
            full((1, LANEPAD)), full((1, LANEPAD)), full((1, LANEPAD)),
            pl.BlockSpec((BM1, H // 2), lambda i: (i, 0)),
        ],
        out_shape=[
            jax.ShapeDtypeStruct((B, E), jnp.float32),
            jax.ShapeDtypeStruct((B, E), jnp.float32),
            jax.ShapeDtypeStruct((B, 1), jnp.float32),
            jax.ShapeDtypeStruct((B, 1), jnp.float32),
            jax.ShapeDtypeStruct((4, B), jnp.int32),
            jax.ShapeDtypeStruct((1, LANEPAD), jnp.int32),
            jax.ShapeDtypeStruct((1, LANEPAD), jnp.int32),
            jax.ShapeDtypeStruct((1, LANEPAD), jnp.int32),
            jax.ShapeDtypeStruct((B, H // 2), jnp.int32),
        ],
        scratch_shapes=[pltpu.VMEM((1, E), jnp.float32)],
        compiler_params=pltpu.CompilerParams(
            dimension_semantics=("arbitrary",)),
    )(x, gW1, gb1.reshape(1, HG), gln_g.reshape(1, HG), gln_b.reshape(1, HG),
      gW2, gb2.reshape(1, E))

    xs, pos1, pos2 = _stage2_dispatch(counts, rt, xp)

    z = pl.pallas_call(
        _gmm_body,
        grid_spec=pltpu.PrefetchScalarGridSpec(
            num_scalar_prefetch=2,
            grid=(NBLK,),
            in_specs=[
                pl.BlockSpec((BMG, H // 2), lambda j, bg, bv: (bv[0, j] * j, 0)),
                pl.BlockSpec((1, H, H), lambda j, bg, bv: (bg[0, j], 0, 0)),
                pl.BlockSpec((H, HG), lambda j, bg, bv: (0, 0)),
            ],
            out_specs=pl.BlockSpec(
                (BMG, HG // 2), lambda j, bg, bv: (jnp.where(bv[0, j] == 1, j, NBLK), 0)),
            scratch_shapes=[pltpu.VMEM((H, HG), jnp.bfloat16)],
        ),
        out_shape=jax.ShapeDtypeStruct(((NBLK + 1) * BMG, HG // 2), jnp.int32),
        compiler_params=pltpu.CompilerParams(
            dimension_semantics=("arbitrary",)),
    )(bg, bvalid, xs, eW, cW1)

    zg0, zg1 = _stage4_combine(pos1, pos2, z)

    nb5 = B // BM5
    logits = pl.pallas_call(
        _classifier_body,
        grid=(nb5,),
        in_specs=[
            pl.BlockSpec((BM5, HG // 2), lambda i: (i, 0)),
            pl.BlockSpec((BM5, HG // 2), lambda i: (i, 0)),
            pl.BlockSpec((BM5, 1), lambda i: (i, 0)),
            pl.BlockSpec((BM5, 1), lambda i: (i, 0)),
            pl.BlockSpec((BM5, E), lambda i: (i, 0)),
            full((E, H)),
            full((H, HG)), full((1, HG)), full((1, HG)), full((1, HG)),
            full((HG, C)), full((1, C)),
        ],
        out_specs=pl.BlockSpec((BM5, C), lambda i: (i, 0)),
        out_shape=jax.ShapeDtypeStruct((B, C), jnp.float32),
        compiler_params=pltpu.CompilerParams(
            dimension_semantics=("parallel",)),
    )(zg0, zg1, w1, w2, combine, eb,
      cW1, cb1.reshape(1, HG), cln_g.reshape(1, HG), cln_b.reshape(1, HG),
      cW2, cb2.reshape(1, C))

    return logits, probs
